# baseline (speedup 1.0000x reference)
import functools

import jax
import jax.numpy as jnp
from jax import lax
from jax.experimental import pallas as pl
from jax.experimental.pallas import tpu as pltpu
from jax.experimental.pallas import tpu_sc as plsc

B, L = 1024, 200
NUCL_DIM, SPEC_DIM = 128, 64
OUT_DIM = NUCL_DIM + 2 * SPEC_DIM
NW = 32
BPW = B // NW
NVR = OUT_DIM // 16
LPAD = 208
GROUPS = LPAD // 16
GF = 80


def _body(seq_ref, pbsf_ref, rtf_ref, nucl_ref, pbst_ref, rtt_ref,
          out_ref, ctab_hbm,
          nucl_v, pbst_v, rtt_v, ctab_v, pbsf_v, rtf_v, seq_all, idx_all,
          ga0, ga1, tb0, tb1, sg0, sg1, soa0, soa1, sob0, sob1):
    wid = lax.axis_index("s") * 2 + lax.axis_index("c")
    base = wid * BPW

    pltpu.sync_copy(nucl_ref, nucl_v)
    pltpu.sync_copy(pbst_ref, pbst_v)
    pltpu.sync_copy(rtt_ref, rtt_v)
    for pi in range(2):
        for ri in range(2):
            for v in range(6):
                row = 12 * pi + 6 * ri + v
                for k in range(NUCL_DIM // 16):
                    ctab_v[row, pl.ds(16 * k, 16)] = nucl_v[v, pl.ds(16 * k, 16)]
                for k in range(SPEC_DIM // 16):
                    ctab_v[row, pl.ds(NUCL_DIM + 16 * k, 16)] = pbst_v[pi, pl.ds(16 * k, 16)]
                for k in range(SPEC_DIM // 16):
                    ctab_v[row, pl.ds(NUCL_DIM + SPEC_DIM + 16 * k, 16)] = rtt_v[ri, pl.ds(16 * k, 16)]
    pltpu.sync_copy(ctab_v, ctab_hbm.at[pl.ds(wid * 24, 24)])

    pltpu.sync_copy(pbsf_ref.at[pl.ds(base, BPW)], pbsf_v)
    pltpu.sync_copy(rtf_ref.at[pl.ds(base, BPW)], rtf_v)
    half = jnp.full((16,), 0.5, jnp.float32)
    combos = []
    for k in range(BPW // 16):
        pv = pbsf_v[pl.ds(16 * k, 16)]
        rv = rtf_v[pl.ds(16 * k, 16)]
        combo = jnp.where(pv > half, jnp.int32(12), jnp.int32(0))
        combos.append(combo + jnp.where(rv > half, jnp.int32(6), jnp.int32(0)))

    seq_all[pl.ds(BPW * L, 16)] = jnp.zeros((16,), jnp.int32)
    pltpu.sync_copy(seq_ref.at[pl.ds(base * L, BPW * L)], seq_all.at[pl.ds(0, BPW * L)])

    for j in range(BPW):
        offj = combos[j // 16][j % 16] + wid * 24
        for k in range(GF // 16):
            idx_all[j, pl.ds(16 * k, 16)] = seq_all[pl.ds(j * L + 16 * k, 16)] + offj

    def build_rows(j, tb):
        in_lo = jnp.full((16,), j < 16)
        cvec = jnp.where(in_lo, combos[0], combos[1])
        lane_ids = lax.iota(jnp.int32, 16)
        off = jnp.sum(jnp.where(lane_ids == (j % 16), cvec, jnp.int32(0)))

        def grp(g, carry):
            rowvec = seq_all[pl.ds(j * L + GF + 16 * g, 16)] + off
            rows = [rowvec[dr] for dr in range(16)]
            for dr in range(16):
                for k in range(NVR):
                    tb[16 * g + dr, pl.ds(16 * k, 16)] = ctab_v[rows[dr], pl.ds(16 * k, 16)]
            return carry

        lax.fori_loop(0, GROUPS - GF // 16, grp, 0)

    gas, tbs = (ga0, ga1), (tb0, tb1)
    sgs, soas, sobs = (sg0, sg1), (soa0, soa1), (sob0, sob1)

    def pair(p, carry):
        for parity in range(2):
            j = 2 * p + parity
            ga, tb = gas[parity], tbs[parity]
            sg, soa, sob = sgs[parity], soas[parity], sobs[parity]

            @pl.when(p >= 1)
            def _wait_prev():
                pltpu.make_async_copy(
                    ga, out_ref.at[pl.ds(base * L, GF)], soa).wait()
                pltpu.make_async_copy(
                    tb.at[pl.ds(0, L - GF)],
                    out_ref.at[pl.ds(base * L, L - GF)], sob).wait()

            gh = pltpu.async_copy(ctab_hbm.at[idx_all.at[j, pl.ds(0, GF)]],
                                  ga, sg)
            build_rows(j, tb)
            pltpu.async_copy(tb.at[pl.ds(0, L - GF)],
                             out_ref.at[pl.ds((base + j) * L + GF, L - GF)], sob)
            gh.wait()
            pltpu.async_copy(ga, out_ref.at[pl.ds((base + j) * L, GF)], soa)
        return carry

    lax.fori_loop(0, BPW // 2, pair, 0)
    for parity in range(2):
        pltpu.make_async_copy(
            gas[parity], out_ref.at[pl.ds(base * L, GF)], soas[parity]).wait()
        pltpu.make_async_copy(
            tbs[parity].at[pl.ds(0, L - GF)],
            out_ref.at[pl.ds(base * L, L - GF)], sobs[parity]).wait()


def kernel(seq, pbs_feat, rt_feat, nucl_table, pbs_table, rt_table):
    mesh = plsc.VectorSubcoreMesh(core_axis_name="c", subcore_axis_name="s")
    run = functools.partial(
        pl.kernel,
        mesh=mesh,
        compiler_params=pltpu.CompilerParams(needs_layout_passes=False),
        out_type=[
            jax.ShapeDtypeStruct((B * L, OUT_DIM), jnp.float32),
            jax.ShapeDtypeStruct((NW * 24, OUT_DIM), jnp.float32),
        ],
        scratch_types=[
            pltpu.VMEM((6, NUCL_DIM), jnp.float32),
            pltpu.VMEM((2, SPEC_DIM), jnp.float32),
            pltpu.VMEM((2, SPEC_DIM), jnp.float32),
            pltpu.VMEM((24, OUT_DIM), jnp.float32),
            pltpu.VMEM((BPW,), jnp.float32),
            pltpu.VMEM((BPW,), jnp.float32),
            pltpu.VMEM((BPW * L + 16,), jnp.int32),
            pltpu.VMEM((BPW, GF), jnp.int32),
            pltpu.VMEM((GF, OUT_DIM), jnp.float32),
            pltpu.VMEM((GF, OUT_DIM), jnp.float32),
            pltpu.VMEM((LPAD - GF, OUT_DIM), jnp.float32),
            pltpu.VMEM((LPAD - GF, OUT_DIM), jnp.float32),
            pltpu.SemaphoreType.DMA,
            pltpu.SemaphoreType.DMA,
            pltpu.SemaphoreType.DMA,
            pltpu.SemaphoreType.DMA,
            pltpu.SemaphoreType.DMA,
            pltpu.SemaphoreType.DMA,
        ],
    )(_body)
    out, _ = run(seq.reshape(B * L), pbs_feat, rt_feat,
                 nucl_table, pbs_table, rt_table)
    return out.reshape(B, L, OUT_DIM)


# speedup vs baseline: 1.1753x; 1.1753x over previous
import functools

import jax
import jax.numpy as jnp
from jax import lax
from jax.experimental import pallas as pl
from jax.experimental.pallas import tpu as pltpu
from jax.experimental.pallas import tpu_sc as plsc

B, L = 1024, 200
NUCL_DIM, SPEC_DIM = 128, 64
OUT_DIM = NUCL_DIM + 2 * SPEC_DIM
NW = 32
BPW = B // NW
NVR = OUT_DIM // 16
LPAD = 208
GROUPS = LPAD // 16
GF = 96


def _body(seq_ref, pbsf_ref, rtf_ref, nucl_ref, pbst_ref, rtt_ref,
          out_ref, ctab_hbm,
          nucl_v, pbst_v, rtt_v, ctab_v, pbsf_v, rtf_v, seq_all, idx_all,
          ga0, ga1, tb0, tb1, sg0, sg1, soa0, soa1, sob0, sob1):
    wid = lax.axis_index("s") * 2 + lax.axis_index("c")
    base = wid * BPW

    pltpu.sync_copy(nucl_ref, nucl_v)
    pltpu.sync_copy(pbst_ref, pbst_v)
    pltpu.sync_copy(rtt_ref, rtt_v)
    for pi in range(2):
        for ri in range(2):
            for v in range(6):
                row = 12 * pi + 6 * ri + v
                for k in range(NUCL_DIM // 16):
                    ctab_v[row, pl.ds(16 * k, 16)] = nucl_v[v, pl.ds(16 * k, 16)]
                for k in range(SPEC_DIM // 16):
                    ctab_v[row, pl.ds(NUCL_DIM + 16 * k, 16)] = pbst_v[pi, pl.ds(16 * k, 16)]
                for k in range(SPEC_DIM // 16):
                    ctab_v[row, pl.ds(NUCL_DIM + SPEC_DIM + 16 * k, 16)] = rtt_v[ri, pl.ds(16 * k, 16)]
    pltpu.sync_copy(ctab_v, ctab_hbm.at[pl.ds(wid * 24, 24)])

    pltpu.sync_copy(pbsf_ref.at[pl.ds(base, BPW)], pbsf_v)
    pltpu.sync_copy(rtf_ref.at[pl.ds(base, BPW)], rtf_v)
    half = jnp.full((16,), 0.5, jnp.float32)
    combos = []
    for k in range(BPW // 16):
        pv = pbsf_v[pl.ds(16 * k, 16)]
        rv = rtf_v[pl.ds(16 * k, 16)]
        combo = jnp.where(pv > half, jnp.int32(12), jnp.int32(0))
        combos.append(combo + jnp.where(rv > half, jnp.int32(6), jnp.int32(0)))

    seq_all[pl.ds(BPW * L, 16)] = jnp.zeros((16,), jnp.int32)
    pltpu.sync_copy(seq_ref.at[pl.ds(base * L, BPW * L)], seq_all.at[pl.ds(0, BPW * L)])

    for j in range(BPW):
        offj = combos[j // 16][j % 16] + wid * 24
        for k in range(GF // 16):
            idx_all[j, pl.ds(16 * k, 16)] = seq_all[pl.ds(j * L + 16 * k, 16)] + offj

    def build_rows(j, tb):
        in_lo = jnp.full((16,), j < 16)
        cvec = jnp.where(in_lo, combos[0], combos[1])
        lane_ids = lax.iota(jnp.int32, 16)
        off = jnp.sum(jnp.where(lane_ids == (j % 16), cvec, jnp.int32(0)))

        def grp(g, carry):
            rowvec = seq_all[pl.ds(j * L + GF + 16 * g, 16)] + off
            rows = [rowvec[dr] for dr in range(16)]
            for dr in range(16):
                vals = [ctab_v[rows[dr], pl.ds(16 * k, 16)] for k in range(NVR)]
                for k in range(NVR):
                    tb[16 * g + dr, pl.ds(16 * k, 16)] = vals[k]
            return carry

        lax.fori_loop(0, GROUPS - GF // 16, grp, 0)

    gas, tbs = (ga0, ga1), (tb0, tb1)
    sgs, soas, sobs = (sg0, sg1), (soa0, soa1), (sob0, sob1)

    def pair(p, carry):
        for parity in range(2):
            j = 2 * p + parity
            ga, tb = gas[parity], tbs[parity]
            sg, soa, sob = sgs[parity], soas[parity], sobs[parity]

            @pl.when(p >= 1)
            def _wait_prev():
                pltpu.make_async_copy(
                    ga, out_ref.at[pl.ds(base * L, GF)], soa).wait()
                pltpu.make_async_copy(
                    tb.at[pl.ds(0, L - GF)],
                    out_ref.at[pl.ds(base * L, L - GF)], sob).wait()

            gh = pltpu.async_copy(ctab_hbm.at[idx_all.at[j, pl.ds(0, GF)]],
                                  ga, sg)
            build_rows(j, tb)
            pltpu.async_copy(tb.at[pl.ds(0, L - GF)],
                             out_ref.at[pl.ds((base + j) * L + GF, L - GF)], sob)
            gh.wait()
            pltpu.async_copy(ga, out_ref.at[pl.ds((base + j) * L, GF)], soa)
        return carry

    lax.fori_loop(0, BPW // 2, pair, 0)
    for parity in range(2):
        pltpu.make_async_copy(
            gas[parity], out_ref.at[pl.ds(base * L, GF)], soas[parity]).wait()
        pltpu.make_async_copy(
            tbs[parity].at[pl.ds(0, L - GF)],
            out_ref.at[pl.ds(base * L, L - GF)], sobs[parity]).wait()


def kernel(seq, pbs_feat, rt_feat, nucl_table, pbs_table, rt_table):
    mesh = plsc.VectorSubcoreMesh(core_axis_name="c", subcore_axis_name="s")
    run = functools.partial(
        pl.kernel,
        mesh=mesh,
        compiler_params=pltpu.CompilerParams(needs_layout_passes=False),
        out_type=[
            jax.ShapeDtypeStruct((B * L, OUT_DIM), jnp.float32),
            jax.ShapeDtypeStruct((NW * 24, OUT_DIM), jnp.float32),
        ],
        scratch_types=[
            pltpu.VMEM((6, NUCL_DIM), jnp.float32),
            pltpu.VMEM((2, SPEC_DIM), jnp.float32),
            pltpu.VMEM((2, SPEC_DIM), jnp.float32),
            pltpu.VMEM((24, OUT_DIM), jnp.float32),
            pltpu.VMEM((BPW,), jnp.float32),
            pltpu.VMEM((BPW,), jnp.float32),
            pltpu.VMEM((BPW * L + 16,), jnp.int32),
            pltpu.VMEM((BPW, GF), jnp.int32),
            pltpu.VMEM((GF, OUT_DIM), jnp.float32),
            pltpu.VMEM((GF, OUT_DIM), jnp.float32),
            pltpu.VMEM((LPAD - GF, OUT_DIM), jnp.float32),
            pltpu.VMEM((LPAD - GF, OUT_DIM), jnp.float32),
            pltpu.SemaphoreType.DMA,
            pltpu.SemaphoreType.DMA,
            pltpu.SemaphoreType.DMA,
            pltpu.SemaphoreType.DMA,
            pltpu.SemaphoreType.DMA,
            pltpu.SemaphoreType.DMA,
        ],
    )(_body)
    out, _ = run(seq.reshape(B * L), pbs_feat, rt_feat,
                 nucl_table, pbs_table, rt_table)
    return out.reshape(B, L, OUT_DIM)


# speedup vs baseline: 1.2931x; 1.1002x over previous
import functools

import jax
import jax.numpy as jnp
from jax import lax
from jax.experimental import pallas as pl
from jax.experimental.pallas import tpu as pltpu
from jax.experimental.pallas import tpu_sc as plsc

B, L = 1024, 200
NUCL_DIM, SPEC_DIM = 128, 64
OUT_DIM = NUCL_DIM + 2 * SPEC_DIM
NW = 32
BPW = B // NW
NVR = OUT_DIM // 16
LPAD = 208
GROUPS = LPAD // 16
GF = 80


def _body(seq_ref, pbsf_ref, rtf_ref, nucl_ref, pbst_ref, rtt_ref,
          out_ref, ctab_hbm,
          nucl_v, pbst_v, rtt_v, ctab_v, pbsf_v, rtf_v, seq_all, idx_all,
          ga0, ga1, tb0, tb1, sg0, sg1, soa0, soa1, sob0, sob1):
    wid = lax.axis_index("s") * 2 + lax.axis_index("c")
    base = wid * BPW

    pltpu.sync_copy(nucl_ref, nucl_v)
    pltpu.sync_copy(pbst_ref, pbst_v)
    pltpu.sync_copy(rtt_ref, rtt_v)
    for pi in range(2):
        for ri in range(2):
            for v in range(6):
                row = 12 * pi + 6 * ri + v
                for k in range(NUCL_DIM // 16):
                    ctab_v[row, pl.ds(16 * k, 16)] = nucl_v[v, pl.ds(16 * k, 16)]
                for k in range(SPEC_DIM // 16):
                    ctab_v[row, pl.ds(NUCL_DIM + 16 * k, 16)] = pbst_v[pi, pl.ds(16 * k, 16)]
                for k in range(SPEC_DIM // 16):
                    ctab_v[row, pl.ds(NUCL_DIM + SPEC_DIM + 16 * k, 16)] = rtt_v[ri, pl.ds(16 * k, 16)]
    pltpu.sync_copy(ctab_v, ctab_hbm.at[pl.ds(wid * 24, 24)])

    pltpu.sync_copy(pbsf_ref.at[pl.ds(base, BPW)], pbsf_v)
    pltpu.sync_copy(rtf_ref.at[pl.ds(base, BPW)], rtf_v)
    half = jnp.full((16,), 0.5, jnp.float32)
    combos = []
    for k in range(BPW // 16):
        pv = pbsf_v[pl.ds(16 * k, 16)]
        rv = rtf_v[pl.ds(16 * k, 16)]
        combo = jnp.where(pv > half, jnp.int32(12), jnp.int32(0))
        combos.append(combo + jnp.where(rv > half, jnp.int32(6), jnp.int32(0)))

    seq_all[pl.ds(BPW * L, 16)] = jnp.zeros((16,), jnp.int32)
    pltpu.sync_copy(seq_ref.at[pl.ds(base * L, BPW * L)], seq_all.at[pl.ds(0, BPW * L)])

    for j in range(BPW):
        offj = combos[j // 16][j % 16] + wid * 24
        for k in range(GF // 16):
            idx_all[j, pl.ds(16 * k, 16)] = seq_all[pl.ds(j * L + 16 * k, 16)] + offj

    def build_rows(j, tb):
        in_lo = jnp.full((16,), j < 16)
        cvec = jnp.where(in_lo, combos[0], combos[1])
        lane_ids = lax.iota(jnp.int32, 16)
        off = jnp.sum(jnp.where(lane_ids == (j % 16), cvec, jnp.int32(0)))

        def grp(g, carry):
            rowvec = seq_all[pl.ds(j * L + GF + 16 * g, 16)] + off
            rows = [rowvec[dr] for dr in range(16)]
            for dr in range(16):
                vals = [ctab_v[rows[dr], pl.ds(16 * k, 16)] for k in range(NVR)]
                for k in range(NVR):
                    tb[16 * g + dr, pl.ds(16 * k, 16)] = vals[k]
            return carry

        lax.fori_loop(0, GROUPS - GF // 16, grp, 0)

    gas, tbs = (ga0, ga1), (tb0, tb1)
    sgs, soas, sobs = (sg0, sg1), (soa0, soa1), (sob0, sob1)

    def pair(p, carry):
        for parity in range(2):
            j = 2 * p + parity
            ga, tb = gas[parity], tbs[parity]
            sg, soa, sob = sgs[parity], soas[parity], sobs[parity]

            @pl.when(p >= 1)
            def _wait_prev():
                pltpu.make_async_copy(
                    ga, out_ref.at[pl.ds(base * L, GF)], soa).wait()
                pltpu.make_async_copy(
                    tb.at[pl.ds(0, L - GF)],
                    out_ref.at[pl.ds(base * L, L - GF)], sob).wait()

            gh = pltpu.async_copy(ctab_hbm.at[idx_all.at[j, pl.ds(0, GF)]],
                                  ga, sg)
            build_rows(j, tb)
            pltpu.async_copy(tb.at[pl.ds(0, L - GF)],
                             out_ref.at[pl.ds((base + j) * L + GF, L - GF)], sob)
            gh.wait()
            pltpu.async_copy(ga, out_ref.at[pl.ds((base + j) * L, GF)], soa)
        return carry

    lax.fori_loop(0, BPW // 2, pair, 0)
    for parity in range(2):
        pltpu.make_async_copy(
            gas[parity], out_ref.at[pl.ds(base * L, GF)], soas[parity]).wait()
        pltpu.make_async_copy(
            tbs[parity].at[pl.ds(0, L - GF)],
            out_ref.at[pl.ds(base * L, L - GF)], sobs[parity]).wait()


def kernel(seq, pbs_feat, rt_feat, nucl_table, pbs_table, rt_table):
    mesh = plsc.VectorSubcoreMesh(core_axis_name="c", subcore_axis_name="s")
    run = functools.partial(
        pl.kernel,
        mesh=mesh,
        compiler_params=pltpu.CompilerParams(needs_layout_passes=False),
        out_type=[
            jax.ShapeDtypeStruct((B * L, OUT_DIM), jnp.float32),
            jax.ShapeDtypeStruct((NW * 24, OUT_DIM), jnp.float32),
        ],
        scratch_types=[
            pltpu.VMEM((6, NUCL_DIM), jnp.float32),
            pltpu.VMEM((2, SPEC_DIM), jnp.float32),
            pltpu.VMEM((2, SPEC_DIM), jnp.float32),
            pltpu.VMEM((24, OUT_DIM), jnp.float32),
            pltpu.VMEM((BPW,), jnp.float32),
            pltpu.VMEM((BPW,), jnp.float32),
            pltpu.VMEM((BPW * L + 16,), jnp.int32),
            pltpu.VMEM((BPW, GF), jnp.int32),
            pltpu.VMEM((GF, OUT_DIM), jnp.float32),
            pltpu.VMEM((GF, OUT_DIM), jnp.float32),
            pltpu.VMEM((LPAD - GF, OUT_DIM), jnp.float32),
            pltpu.VMEM((LPAD - GF, OUT_DIM), jnp.float32),
            pltpu.SemaphoreType.DMA,
            pltpu.SemaphoreType.DMA,
            pltpu.SemaphoreType.DMA,
            pltpu.SemaphoreType.DMA,
            pltpu.SemaphoreType.DMA,
            pltpu.SemaphoreType.DMA,
        ],
    )(_body)
    out, _ = run(seq.reshape(B * L), pbs_feat, rt_feat,
                 nucl_table, pbs_table, rt_table)
    return out.reshape(B, L, OUT_DIM)


# speedup vs baseline: 1.4316x; 1.1071x over previous
import functools

import jax
import jax.numpy as jnp
from jax import lax
from jax.experimental import pallas as pl
from jax.experimental.pallas import tpu as pltpu
from jax.experimental.pallas import tpu_sc as plsc

B, L = 1024, 200
NUCL_DIM, SPEC_DIM = 128, 64
OUT_DIM = NUCL_DIM + 2 * SPEC_DIM
NW = 32
BPW = B // NW
NVR = OUT_DIM // 16
LPAD = 208
GROUPS = LPAD // 16
GF = 64


def _body(seq_ref, pbsf_ref, rtf_ref, nucl_ref, pbst_ref, rtt_ref,
          out_ref, ctab_hbm,
          nucl_v, pbst_v, rtt_v, ctab_v, pbsf_v, rtf_v, seq_all, idx_all,
          ga0, ga1, tb0, tb1, sg0, sg1, soa0, soa1, sob0, sob1):
    wid = lax.axis_index("s") * 2 + lax.axis_index("c")
    base = wid * BPW

    pltpu.sync_copy(nucl_ref, nucl_v)
    pltpu.sync_copy(pbst_ref, pbst_v)
    pltpu.sync_copy(rtt_ref, rtt_v)
    for pi in range(2):
        for ri in range(2):
            for v in range(6):
                row = 12 * pi + 6 * ri + v
                for k in range(NUCL_DIM // 16):
                    ctab_v[row, pl.ds(16 * k, 16)] = nucl_v[v, pl.ds(16 * k, 16)]
                for k in range(SPEC_DIM // 16):
                    ctab_v[row, pl.ds(NUCL_DIM + 16 * k, 16)] = pbst_v[pi, pl.ds(16 * k, 16)]
                for k in range(SPEC_DIM // 16):
                    ctab_v[row, pl.ds(NUCL_DIM + SPEC_DIM + 16 * k, 16)] = rtt_v[ri, pl.ds(16 * k, 16)]
    pltpu.sync_copy(ctab_v, ctab_hbm.at[pl.ds(wid * 24, 24)])

    pltpu.sync_copy(pbsf_ref.at[pl.ds(base, BPW)], pbsf_v)
    pltpu.sync_copy(rtf_ref.at[pl.ds(base, BPW)], rtf_v)
    half = jnp.full((16,), 0.5, jnp.float32)
    combos = []
    for k in range(BPW // 16):
        pv = pbsf_v[pl.ds(16 * k, 16)]
        rv = rtf_v[pl.ds(16 * k, 16)]
        combo = jnp.where(pv > half, jnp.int32(12), jnp.int32(0))
        combos.append(combo + jnp.where(rv > half, jnp.int32(6), jnp.int32(0)))

    seq_all[pl.ds(BPW * L, 16)] = jnp.zeros((16,), jnp.int32)
    pltpu.sync_copy(seq_ref.at[pl.ds(base * L, BPW * L)], seq_all.at[pl.ds(0, BPW * L)])

    for j in range(BPW):
        offj = combos[j // 16][j % 16] + wid * 24
        for k in range(GF // 16):
            idx_all[j, pl.ds(16 * k, 16)] = seq_all[pl.ds(j * L + 16 * k, 16)] + offj

    def build_rows(j, tb):
        in_lo = jnp.full((16,), j < 16)
        cvec = jnp.where(in_lo, combos[0], combos[1])
        lane_ids = lax.iota(jnp.int32, 16)
        off = jnp.sum(jnp.where(lane_ids == (j % 16), cvec, jnp.int32(0)))

        def grp(g, carry):
            rowvec = seq_all[pl.ds(j * L + GF + 16 * g, 16)] + off
            rows = [rowvec[dr] for dr in range(16)]
            for dr in range(16):
                vals = [ctab_v[rows[dr], pl.ds(16 * k, 16)] for k in range(NVR)]
                for k in range(NVR):
                    tb[16 * g + dr, pl.ds(16 * k, 16)] = vals[k]
            return carry

        lax.fori_loop(0, GROUPS - GF // 16, grp, 0)

    gas, tbs = (ga0, ga1), (tb0, tb1)
    sgs, soas, sobs = (sg0, sg1), (soa0, soa1), (sob0, sob1)

    def pair(p, carry):
        for parity in range(2):
            j = 2 * p + parity
            ga, tb = gas[parity], tbs[parity]
            sg, soa, sob = sgs[parity], soas[parity], sobs[parity]

            @pl.when(p >= 1)
            def _wait_prev():
                pltpu.make_async_copy(
                    ga, out_ref.at[pl.ds(base * L, GF)], soa).wait()
                pltpu.make_async_copy(
                    tb.at[pl.ds(0, L - GF)],
                    out_ref.at[pl.ds(base * L, L - GF)], sob).wait()

            gh = pltpu.async_copy(ctab_hbm.at[idx_all.at[j, pl.ds(0, GF)]],
                                  ga, sg)
            build_rows(j, tb)
            pltpu.async_copy(tb.at[pl.ds(0, L - GF)],
                             out_ref.at[pl.ds((base + j) * L + GF, L - GF)], sob)
            gh.wait()
            pltpu.async_copy(ga, out_ref.at[pl.ds((base + j) * L, GF)], soa)
        return carry

    lax.fori_loop(0, BPW // 2, pair, 0)
    for parity in range(2):
        pltpu.make_async_copy(
            gas[parity], out_ref.at[pl.ds(base * L, GF)], soas[parity]).wait()
        pltpu.make_async_copy(
            tbs[parity].at[pl.ds(0, L - GF)],
            out_ref.at[pl.ds(base * L, L - GF)], sobs[parity]).wait()


def kernel(seq, pbs_feat, rt_feat, nucl_table, pbs_table, rt_table):
    mesh = plsc.VectorSubcoreMesh(core_axis_name="c", subcore_axis_name="s")
    run = functools.partial(
        pl.kernel,
        mesh=mesh,
        compiler_params=pltpu.CompilerParams(needs_layout_passes=False),
        out_type=[
            jax.ShapeDtypeStruct((B * L, OUT_DIM), jnp.float32),
            jax.ShapeDtypeStruct((NW * 24, OUT_DIM), jnp.float32),
        ],
        scratch_types=[
            pltpu.VMEM((6, NUCL_DIM), jnp.float32),
            pltpu.VMEM((2, SPEC_DIM), jnp.float32),
            pltpu.VMEM((2, SPEC_DIM), jnp.float32),
            pltpu.VMEM((24, OUT_DIM), jnp.float32),
            pltpu.VMEM((BPW,), jnp.float32),
            pltpu.VMEM((BPW,), jnp.float32),
            pltpu.VMEM((BPW * L + 16,), jnp.int32),
            pltpu.VMEM((BPW, GF), jnp.int32),
            pltpu.VMEM((GF, OUT_DIM), jnp.float32),
            pltpu.VMEM((GF, OUT_DIM), jnp.float32),
            pltpu.VMEM((LPAD - GF, OUT_DIM), jnp.float32),
            pltpu.VMEM((LPAD - GF, OUT_DIM), jnp.float32),
            pltpu.SemaphoreType.DMA,
            pltpu.SemaphoreType.DMA,
            pltpu.SemaphoreType.DMA,
            pltpu.SemaphoreType.DMA,
            pltpu.SemaphoreType.DMA,
            pltpu.SemaphoreType.DMA,
        ],
    )(_body)
    out, _ = run(seq.reshape(B * L), pbs_feat, rt_feat,
                 nucl_table, pbs_table, rt_table)
    return out.reshape(B, L, OUT_DIM)


# speedup vs baseline: 1.6247x; 1.1349x over previous
import functools

import jax
import jax.numpy as jnp
from jax import lax
from jax.experimental import pallas as pl
from jax.experimental.pallas import tpu as pltpu
from jax.experimental.pallas import tpu_sc as plsc

B, L = 1024, 200
NUCL_DIM, SPEC_DIM = 128, 64
OUT_DIM = NUCL_DIM + 2 * SPEC_DIM
NW = 32
BPW = B // NW
NVR = OUT_DIM // 16
LPAD = 208
GROUPS = LPAD // 16
GF = 48


def _body(seq_ref, pbsf_ref, rtf_ref, nucl_ref, pbst_ref, rtt_ref,
          out_ref, ctab_hbm,
          nucl_v, pbst_v, rtt_v, ctab_v, pbsf_v, rtf_v, seq_all, idx_all,
          ga0, ga1, tb0, tb1, sg0, sg1, soa0, soa1, sob0, sob1):
    wid = lax.axis_index("s") * 2 + lax.axis_index("c")
    base = wid * BPW

    pltpu.sync_copy(nucl_ref, nucl_v)
    pltpu.sync_copy(pbst_ref, pbst_v)
    pltpu.sync_copy(rtt_ref, rtt_v)
    for pi in range(2):
        for ri in range(2):
            for v in range(6):
                row = 12 * pi + 6 * ri + v
                for k in range(NUCL_DIM // 16):
                    ctab_v[row, pl.ds(16 * k, 16)] = nucl_v[v, pl.ds(16 * k, 16)]
                for k in range(SPEC_DIM // 16):
                    ctab_v[row, pl.ds(NUCL_DIM + 16 * k, 16)] = pbst_v[pi, pl.ds(16 * k, 16)]
                for k in range(SPEC_DIM // 16):
                    ctab_v[row, pl.ds(NUCL_DIM + SPEC_DIM + 16 * k, 16)] = rtt_v[ri, pl.ds(16 * k, 16)]
    pltpu.sync_copy(ctab_v, ctab_hbm.at[pl.ds(wid * 24, 24)])

    pltpu.sync_copy(pbsf_ref.at[pl.ds(base, BPW)], pbsf_v)
    pltpu.sync_copy(rtf_ref.at[pl.ds(base, BPW)], rtf_v)
    half = jnp.full((16,), 0.5, jnp.float32)
    combos = []
    for k in range(BPW // 16):
        pv = pbsf_v[pl.ds(16 * k, 16)]
        rv = rtf_v[pl.ds(16 * k, 16)]
        combo = jnp.where(pv > half, jnp.int32(12), jnp.int32(0))
        combos.append(combo + jnp.where(rv > half, jnp.int32(6), jnp.int32(0)))

    seq_all[pl.ds(BPW * L, 16)] = jnp.zeros((16,), jnp.int32)
    pltpu.sync_copy(seq_ref.at[pl.ds(base * L, BPW * L)], seq_all.at[pl.ds(0, BPW * L)])

    for j in range(BPW):
        offj = combos[j // 16][j % 16] + wid * 24
        for k in range(GF // 16):
            idx_all[j, pl.ds(16 * k, 16)] = seq_all[pl.ds(j * L + 16 * k, 16)] + offj

    def build_rows(j, tb):
        in_lo = jnp.full((16,), j < 16)
        cvec = jnp.where(in_lo, combos[0], combos[1])
        lane_ids = lax.iota(jnp.int32, 16)
        off = jnp.sum(jnp.where(lane_ids == (j % 16), cvec, jnp.int32(0)))

        def grp(g, carry):
            rowvec = seq_all[pl.ds(j * L + GF + 16 * g, 16)] + off
            rows = [rowvec[dr] for dr in range(16)]
            for dr in range(16):
                vals = [ctab_v[rows[dr], pl.ds(16 * k, 16)] for k in range(NVR)]
                for k in range(NVR):
                    tb[16 * g + dr, pl.ds(16 * k, 16)] = vals[k]
            return carry

        lax.fori_loop(0, GROUPS - GF // 16, grp, 0)

    gas, tbs = (ga0, ga1), (tb0, tb1)
    sgs, soas, sobs = (sg0, sg1), (soa0, soa1), (sob0, sob1)

    def pair(p, carry):
        for parity in range(2):
            j = 2 * p + parity
            ga, tb = gas[parity], tbs[parity]
            sg, soa, sob = sgs[parity], soas[parity], sobs[parity]

            @pl.when(p >= 1)
            def _wait_prev():
                pltpu.make_async_copy(
                    ga, out_ref.at[pl.ds(base * L, GF)], soa).wait()
                pltpu.make_async_copy(
                    tb.at[pl.ds(0, L - GF)],
                    out_ref.at[pl.ds(base * L, L - GF)], sob).wait()

            gh = pltpu.async_copy(ctab_hbm.at[idx_all.at[j, pl.ds(0, GF)]],
                                  ga, sg)
            build_rows(j, tb)
            pltpu.async_copy(tb.at[pl.ds(0, L - GF)],
                             out_ref.at[pl.ds((base + j) * L + GF, L - GF)], sob)
            gh.wait()
            pltpu.async_copy(ga, out_ref.at[pl.ds((base + j) * L, GF)], soa)
        return carry

    lax.fori_loop(0, BPW // 2, pair, 0)
    for parity in range(2):
        pltpu.make_async_copy(
            gas[parity], out_ref.at[pl.ds(base * L, GF)], soas[parity]).wait()
        pltpu.make_async_copy(
            tbs[parity].at[pl.ds(0, L - GF)],
            out_ref.at[pl.ds(base * L, L - GF)], sobs[parity]).wait()


def kernel(seq, pbs_feat, rt_feat, nucl_table, pbs_table, rt_table):
    mesh = plsc.VectorSubcoreMesh(core_axis_name="c", subcore_axis_name="s")
    run = functools.partial(
        pl.kernel,
        mesh=mesh,
        compiler_params=pltpu.CompilerParams(needs_layout_passes=False),
        out_type=[
            jax.ShapeDtypeStruct((B * L, OUT_DIM), jnp.float32),
            jax.ShapeDtypeStruct((NW * 24, OUT_DIM), jnp.float32),
        ],
        scratch_types=[
            pltpu.VMEM((6, NUCL_DIM), jnp.float32),
            pltpu.VMEM((2, SPEC_DIM), jnp.float32),
            pltpu.VMEM((2, SPEC_DIM), jnp.float32),
            pltpu.VMEM((24, OUT_DIM), jnp.float32),
            pltpu.VMEM((BPW,), jnp.float32),
            pltpu.VMEM((BPW,), jnp.float32),
            pltpu.VMEM((BPW * L + 16,), jnp.int32),
            pltpu.VMEM((BPW, GF), jnp.int32),
            pltpu.VMEM((GF, OUT_DIM), jnp.float32),
            pltpu.VMEM((GF, OUT_DIM), jnp.float32),
            pltpu.VMEM((LPAD - GF, OUT_DIM), jnp.float32),
            pltpu.VMEM((LPAD - GF, OUT_DIM), jnp.float32),
            pltpu.SemaphoreType.DMA,
            pltpu.SemaphoreType.DMA,
            pltpu.SemaphoreType.DMA,
            pltpu.SemaphoreType.DMA,
            pltpu.SemaphoreType.DMA,
            pltpu.SemaphoreType.DMA,
        ],
    )(_body)
    out, _ = run(seq.reshape(B * L), pbs_feat, rt_feat,
                 nucl_table, pbs_table, rt_table)
    return out.reshape(B, L, OUT_DIM)


# speedup vs baseline: 1.7689x; 1.0888x over previous
import functools

import jax
import jax.numpy as jnp
from jax import lax
from jax.experimental import pallas as pl
from jax.experimental.pallas import tpu as pltpu
from jax.experimental.pallas import tpu_sc as plsc

B, L = 1024, 200
NUCL_DIM, SPEC_DIM = 128, 64
OUT_DIM = NUCL_DIM + 2 * SPEC_DIM
NW = 32
BPW = B // NW
NVR = OUT_DIM // 16
LPAD = 208
GROUPS = LPAD // 16
GF = 32


def _body(seq_ref, pbsf_ref, rtf_ref, nucl_ref, pbst_ref, rtt_ref,
          out_ref, ctab_hbm,
          nucl_v, pbst_v, rtt_v, ctab_v, pbsf_v, rtf_v, seq_all, idx_all,
          ga0, ga1, tb0, tb1, sg0, sg1, soa0, soa1, sob0, sob1):
    wid = lax.axis_index("s") * 2 + lax.axis_index("c")
    base = wid * BPW

    pltpu.sync_copy(nucl_ref, nucl_v)
    pltpu.sync_copy(pbst_ref, pbst_v)
    pltpu.sync_copy(rtt_ref, rtt_v)
    for pi in range(2):
        for ri in range(2):
            for v in range(6):
                row = 12 * pi + 6 * ri + v
                for k in range(NUCL_DIM // 16):
                    ctab_v[row, pl.ds(16 * k, 16)] = nucl_v[v, pl.ds(16 * k, 16)]
                for k in range(SPEC_DIM // 16):
                    ctab_v[row, pl.ds(NUCL_DIM + 16 * k, 16)] = pbst_v[pi, pl.ds(16 * k, 16)]
                for k in range(SPEC_DIM // 16):
                    ctab_v[row, pl.ds(NUCL_DIM + SPEC_DIM + 16 * k, 16)] = rtt_v[ri, pl.ds(16 * k, 16)]
    pltpu.sync_copy(ctab_v, ctab_hbm.at[pl.ds(wid * 24, 24)])

    pltpu.sync_copy(pbsf_ref.at[pl.ds(base, BPW)], pbsf_v)
    pltpu.sync_copy(rtf_ref.at[pl.ds(base, BPW)], rtf_v)
    half = jnp.full((16,), 0.5, jnp.float32)
    combos = []
    for k in range(BPW // 16):
        pv = pbsf_v[pl.ds(16 * k, 16)]
        rv = rtf_v[pl.ds(16 * k, 16)]
        combo = jnp.where(pv > half, jnp.int32(12), jnp.int32(0))
        combos.append(combo + jnp.where(rv > half, jnp.int32(6), jnp.int32(0)))

    seq_all[pl.ds(BPW * L, 16)] = jnp.zeros((16,), jnp.int32)
    pltpu.sync_copy(seq_ref.at[pl.ds(base * L, BPW * L)], seq_all.at[pl.ds(0, BPW * L)])

    for j in range(BPW):
        offj = combos[j // 16][j % 16] + wid * 24
        for k in range(GF // 16):
            idx_all[j, pl.ds(16 * k, 16)] = seq_all[pl.ds(j * L + 16 * k, 16)] + offj

    def build_rows(j, tb):
        in_lo = jnp.full((16,), j < 16)
        cvec = jnp.where(in_lo, combos[0], combos[1])
        lane_ids = lax.iota(jnp.int32, 16)
        off = jnp.sum(jnp.where(lane_ids == (j % 16), cvec, jnp.int32(0)))

        def grp(g, carry):
            rowvec = seq_all[pl.ds(j * L + GF + 16 * g, 16)] + off
            rows = [rowvec[dr] for dr in range(16)]
            for dr in range(16):
                vals = [ctab_v[rows[dr], pl.ds(16 * k, 16)] for k in range(NVR)]
                for k in range(NVR):
                    tb[16 * g + dr, pl.ds(16 * k, 16)] = vals[k]
            return carry

        lax.fori_loop(0, GROUPS - GF // 16, grp, 0)

    gas, tbs = (ga0, ga1), (tb0, tb1)
    sgs, soas, sobs = (sg0, sg1), (soa0, soa1), (sob0, sob1)

    def pair(p, carry):
        for parity in range(2):
            j = 2 * p + parity
            ga, tb = gas[parity], tbs[parity]
            sg, soa, sob = sgs[parity], soas[parity], sobs[parity]

            @pl.when(p >= 1)
            def _wait_prev():
                pltpu.make_async_copy(
                    ga, out_ref.at[pl.ds(base * L, GF)], soa).wait()
                pltpu.make_async_copy(
                    tb.at[pl.ds(0, L - GF)],
                    out_ref.at[pl.ds(base * L, L - GF)], sob).wait()

            gh = pltpu.async_copy(ctab_hbm.at[idx_all.at[j, pl.ds(0, GF)]],
                                  ga, sg)
            build_rows(j, tb)
            pltpu.async_copy(tb.at[pl.ds(0, L - GF)],
                             out_ref.at[pl.ds((base + j) * L + GF, L - GF)], sob)
            gh.wait()
            pltpu.async_copy(ga, out_ref.at[pl.ds((base + j) * L, GF)], soa)
        return carry

    lax.fori_loop(0, BPW // 2, pair, 0)
    for parity in range(2):
        pltpu.make_async_copy(
            gas[parity], out_ref.at[pl.ds(base * L, GF)], soas[parity]).wait()
        pltpu.make_async_copy(
            tbs[parity].at[pl.ds(0, L - GF)],
            out_ref.at[pl.ds(base * L, L - GF)], sobs[parity]).wait()


def kernel(seq, pbs_feat, rt_feat, nucl_table, pbs_table, rt_table):
    mesh = plsc.VectorSubcoreMesh(core_axis_name="c", subcore_axis_name="s")
    run = functools.partial(
        pl.kernel,
        mesh=mesh,
        compiler_params=pltpu.CompilerParams(needs_layout_passes=False),
        out_type=[
            jax.ShapeDtypeStruct((B * L, OUT_DIM), jnp.float32),
            jax.ShapeDtypeStruct((NW * 24, OUT_DIM), jnp.float32),
        ],
        scratch_types=[
            pltpu.VMEM((6, NUCL_DIM), jnp.float32),
            pltpu.VMEM((2, SPEC_DIM), jnp.float32),
            pltpu.VMEM((2, SPEC_DIM), jnp.float32),
            pltpu.VMEM((24, OUT_DIM), jnp.float32),
            pltpu.VMEM((BPW,), jnp.float32),
            pltpu.VMEM((BPW,), jnp.float32),
            pltpu.VMEM((BPW * L + 16,), jnp.int32),
            pltpu.VMEM((BPW, GF), jnp.int32),
            pltpu.VMEM((GF, OUT_DIM), jnp.float32),
            pltpu.VMEM((GF, OUT_DIM), jnp.float32),
            pltpu.VMEM((LPAD - GF, OUT_DIM), jnp.float32),
            pltpu.VMEM((LPAD - GF, OUT_DIM), jnp.float32),
            pltpu.SemaphoreType.DMA,
            pltpu.SemaphoreType.DMA,
            pltpu.SemaphoreType.DMA,
            pltpu.SemaphoreType.DMA,
            pltpu.SemaphoreType.DMA,
            pltpu.SemaphoreType.DMA,
        ],
    )(_body)
    out, _ = run(seq.reshape(B * L), pbs_feat, rt_feat,
                 nucl_table, pbs_table, rt_table)
    return out.reshape(B, L, OUT_DIM)
